# baseline (speedup 1.0000x reference)
import functools

import jax
import jax.numpy as jnp
from jax import lax
from jax.experimental import pallas as pl
from jax.experimental.pallas import tpu as pltpu
from jax.experimental.pallas import tpu_sc as plsc

NC = 2
NS = 16
NW = NC * NS
LANES = 16
CH = 112
NBUF = 3


def _prep_body(x_ref, w_ref, asr_ref, adr_ref, h_ref, as_ref, ad_ref, m_ref):
    h = jnp.dot(x_ref[...], w_ref[...], preferred_element_type=jnp.float32)
    h_ref[...] = h
    a_s = jnp.sum(h * asr_ref[...], axis=1, keepdims=True)
    a_d = jnp.sum(h * adr_ref[...], axis=1, keepdims=True)
    as_ref[...] = a_s
    ad_ref[...] = a_d

    @pl.when(pl.program_id(0) == 0)
    def _():
        m_ref[0, 0] = -jnp.inf
        m_ref[0, 1] = -jnp.inf

    m_ref[0, 0] = jnp.maximum(m_ref[0, 0], jnp.max(a_s))
    m_ref[0, 1] = jnp.maximum(m_ref[0, 1], jnp.max(a_d))


def _mid_body(acc_ref, den_ref, b_ref, w_ref, asr_ref, adr_ref,
              h_ref, as_ref, ad_ref, m_ref):
    a = acc_ref[0] + acc_ref[1]
    d = den_ref[0] + den_ref[1] + 1e-16
    out1 = a / d + b_ref[...]
    z = jnp.where(out1 > 0, out1, jnp.exp(jnp.minimum(out1, 0.0)) - 1.0)
    h = jnp.dot(z, w_ref[...], preferred_element_type=jnp.float32)
    h_ref[...] = h
    a_s = jnp.sum(h * asr_ref[...], axis=1, keepdims=True)
    a_d = jnp.sum(h * adr_ref[...], axis=1, keepdims=True)
    as_ref[...] = a_s
    ad_ref[...] = a_d

    @pl.when(pl.program_id(0) == 0)
    def _():
        m_ref[0, 0] = -jnp.inf
        m_ref[0, 1] = -jnp.inf

    m_ref[0, 0] = jnp.maximum(m_ref[0, 0], jnp.max(a_s))
    m_ref[0, 1] = jnp.maximum(m_ref[0, 1], jnp.max(a_d))


def _fin_body(acc_ref, den_ref, b_ref, out_ref):
    a = acc_ref[0] + acc_ref[1]
    d = den_ref[0] + den_ref[1] + 1e-16
    out_ref[...] = a / d + b_ref[...]


def _tc_prep(x, W, a_src, a_dst, blk):
    n, dh = x.shape[0], W.shape[1]
    grid = n // blk
    return pl.pallas_call(
        _prep_body,
        grid=(grid,),
        in_specs=[
            pl.BlockSpec((blk, x.shape[1]), lambda i: (i, 0)),
            pl.BlockSpec(W.shape, lambda i: (0, 0)),
            pl.BlockSpec(a_src.shape, lambda i: (0, 0)),
            pl.BlockSpec(a_dst.shape, lambda i: (0, 0)),
        ],
        out_specs=[
            pl.BlockSpec((blk, dh), lambda i: (i, 0)),
            pl.BlockSpec((blk, 1), lambda i: (i, 0)),
            pl.BlockSpec((blk, 1), lambda i: (i, 0)),
            pl.BlockSpec((1, 2), lambda i: (0, 0), memory_space=pltpu.SMEM),
        ],
        out_shape=[
            jax.ShapeDtypeStruct((n, dh), jnp.float32),
            jax.ShapeDtypeStruct((n, 1), jnp.float32),
            jax.ShapeDtypeStruct((n, 1), jnp.float32),
            jax.ShapeDtypeStruct((1, 2), jnp.float32),
        ],
    )(x, W, a_src, a_dst)


def _tc_mid(acc, den, b, W, a_src, a_dst, n, blk):
    dh = W.shape[1]
    grid = n // blk
    return pl.pallas_call(
        _mid_body,
        grid=(grid,),
        in_specs=[
            pl.BlockSpec((2, blk, acc.shape[2]), lambda i: (0, i, 0)),
            pl.BlockSpec((2, blk, 1), lambda i: (0, i, 0)),
            pl.BlockSpec((1, b.shape[1]), lambda i: (0, 0)),
            pl.BlockSpec(W.shape, lambda i: (0, 0)),
            pl.BlockSpec(a_src.shape, lambda i: (0, 0)),
            pl.BlockSpec(a_dst.shape, lambda i: (0, 0)),
        ],
        out_specs=[
            pl.BlockSpec((blk, dh), lambda i: (i, 0)),
            pl.BlockSpec((blk, 1), lambda i: (i, 0)),
            pl.BlockSpec((blk, 1), lambda i: (i, 0)),
            pl.BlockSpec((1, 2), lambda i: (0, 0), memory_space=pltpu.SMEM),
        ],
        out_shape=[
            jax.ShapeDtypeStruct((n, dh), jnp.float32),
            jax.ShapeDtypeStruct((n, 1), jnp.float32),
            jax.ShapeDtypeStruct((n, 1), jnp.float32),
            jax.ShapeDtypeStruct((1, 2), jnp.float32),
        ],
    )(acc, den, b, W, a_src, a_dst)


def _tc_fin(acc, den, b, n, blk):
    dh = acc.shape[2]
    grid = n // blk
    return pl.pallas_call(
        _fin_body,
        grid=(grid,),
        in_specs=[
            pl.BlockSpec((2, blk, dh), lambda i: (0, i, 0)),
            pl.BlockSpec((2, blk, 1), lambda i: (0, i, 0)),
            pl.BlockSpec((1, b.shape[1]), lambda i: (0, 0)),
        ],
        out_specs=pl.BlockSpec((blk, dh), lambda i: (i, 0)),
        out_shape=jax.ShapeDtypeStruct((n, dh), jnp.float32),
    )(acc, den, b)


@functools.lru_cache(maxsize=None)
def _make_sc_edge_kernel(n_pad, dh, chunks):
    mesh = plsc.VectorSubcoreMesh(
        core_axis_name="c", subcore_axis_name="s",
        num_cores=NC, num_subcores=NS)
    rows_per_tile = n_pad // NS
    zfull = rows_per_tile // CH
    zrem = rows_per_tile - zfull * CH

    @functools.partial(
        pl.kernel,
        out_type=(
            jax.ShapeDtypeStruct((NC, n_pad, dh), jnp.float32),
            jax.ShapeDtypeStruct((NC * n_pad,), jnp.float32),
        ),
        mesh=mesh,
        compiler_params=pltpu.CompilerParams(needs_layout_passes=False),
        scratch_types=[
            pltpu.VMEM((NBUF, CH), jnp.int32),
            pltpu.VMEM((NBUF, CH), jnp.int32),
            pltpu.VMEM((NBUF, CH), jnp.float32),
            pltpu.VMEM((NBUF, CH), jnp.float32),
            pltpu.VMEM((LANES,), jnp.float32),
            pltpu.VMEM((NBUF, CH, dh), jnp.float32),
            pltpu.VMEM((CH,), jnp.float32),
            pltpu.VMEM_SHARED((n_pad, dh), jnp.float32),
            pltpu.VMEM_SHARED((n_pad,), jnp.float32),
        ] + [pltpu.SemaphoreType.DMA] * (2 * NBUF),
    )
    def sc_edge(src_hbm, dst_hbm, as_hbm, ad_hbm, m_hbm, h_hbm,
                acc_out, den_out,
                src_i, dst_i, asg_v, adg_v, m_v, rows_v, w_v,
                acc_sh, den_sh, *sems):
        ci = lax.axis_index("c")
        si = lax.axis_index("s")
        wid = ci * NS + si
        semg = sems[:NBUF]
        semi = sems[NBUF:]

        pltpu.sync_copy(m_hbm, m_v)

        zero16 = jnp.zeros((LANES,), jnp.float32)

        def zrow(j, _):
            for r in range(dh // LANES):
                rows_v[0, j, pl.ds(r * LANES, LANES)] = zero16
            return _

        lax.fori_loop(0, CH, zrow, None)
        for r in range(CH // LANES):
            w_v[pl.ds(r * LANES, LANES)] = zero16
        base = si * rows_per_tile
        for t in range(zfull):
            pltpu.sync_copy(rows_v.at[0], acc_sh.at[pl.ds(base + t * CH, CH)])
            pltpu.sync_copy(w_v, den_sh.at[pl.ds(base + t * CH, CH)])
        if zrem:
            rs = pl.ds(base + zfull * CH, zrem)
            pltpu.sync_copy(rows_v.at[0, pl.ds(0, zrem)], acc_sh.at[rs])
            pltpu.sync_copy(w_v.at[pl.ds(0, zrem)], den_sh.at[rs])
        plsc.subcore_barrier()

        m_vec = m_v[...]

        for k in range(NBUF - 1):
            pltpu.sync_copy(src_hbm.at[wid, k], src_i.at[k])
            pltpu.sync_copy(dst_hbm.at[wid, k], dst_i.at[k])
            pltpu.async_copy(h_hbm.at[src_i.at[k]], rows_v.at[k], semg[k])
            pltpu.async_copy(as_hbm.at[src_i.at[k]], asg_v.at[k], semg[k])
            pltpu.async_copy(ad_hbm.at[dst_i.at[k]], adg_v.at[k], semg[k])
        pltpu.async_copy(src_hbm.at[wid, NBUF - 1], src_i.at[NBUF - 1],
                         semi[NBUF - 1])
        pltpu.async_copy(dst_hbm.at[wid, NBUF - 1], dst_i.at[NBUF - 1],
                         semi[NBUF - 1])

        def compute_w(b):
            for g in range(CH // LANES):
                sl = pl.ds(g * LANES, LANES)
                e = asg_v[b, sl] + adg_v[b, sl]
                e = jnp.where(e >= 0, e, e * 0.2) - m_vec
                w_v[sl] = jnp.exp(e)

        def group(g, _):
            for b in range(NBUF):
                j = g * NBUF + b
                t = (b + NBUF - 1) % NBUF

                @pl.when(j + NBUF - 1 < chunks)
                def _():
                    pltpu.make_async_copy(
                        src_hbm.at[wid, j + NBUF - 1], src_i.at[t],
                        semi[t]).wait()
                    pltpu.make_async_copy(
                        dst_hbm.at[wid, j + NBUF - 1], dst_i.at[t],
                        semi[t]).wait()
                    pltpu.async_copy(
                        h_hbm.at[src_i.at[t]], rows_v.at[t], semg[t])
                    pltpu.async_copy(
                        as_hbm.at[src_i.at[t]], asg_v.at[t], semg[t])
                    pltpu.async_copy(
                        ad_hbm.at[dst_i.at[t]], adg_v.at[t], semg[t])

                pltpu.make_async_copy(
                    h_hbm.at[src_i.at[b]], rows_v.at[b], semg[b]).wait()
                pltpu.make_async_copy(
                    as_hbm.at[src_i.at[b]], asg_v.at[b], semg[b]).wait()
                pltpu.make_async_copy(
                    ad_hbm.at[dst_i.at[b]], adg_v.at[b], semg[b]).wait()
                compute_w(b)

                def scale(g2, _c):
                    w16 = w_v[pl.ds(g2 * LANES, LANES)]
                    for jj in range(LANES):
                        w = jnp.full((LANES,), w16[jj])
                        for r in range(dh // LANES):
                            sl = pl.ds(r * LANES, LANES)
                            rows_v[b, g2 * LANES + jj, sl] = (
                                rows_v[b, g2 * LANES + jj, sl] * w)
                    return _c

                lax.fori_loop(0, CH // LANES, scale, None)
                pltpu.sync_copy(rows_v.at[b], acc_sh.at[dst_i.at[b]], add=True)
                pltpu.sync_copy(w_v, den_sh.at[dst_i.at[b]], add=True)

                @pl.when(j + NBUF < chunks)
                def _():
                    pltpu.async_copy(src_hbm.at[wid, j + NBUF], src_i.at[b],
                                     semi[b])
                    pltpu.async_copy(dst_hbm.at[wid, j + NBUF], dst_i.at[b],
                                     semi[b])
            return _

        lax.fori_loop(0, chunks // NBUF, group, None)
        plsc.subcore_barrier()

        dbase = pl.multiple_of(ci * n_pad + base, 8)
        for t in range(zfull):
            rs = pl.ds(base + t * CH, CH)
            pltpu.sync_copy(acc_sh.at[rs], rows_v.at[0])
            pltpu.sync_copy(rows_v.at[0], acc_out.at[ci, rs])
            pltpu.sync_copy(den_sh.at[rs], w_v)
            pltpu.sync_copy(w_v, den_out.at[pl.ds(dbase + t * CH, CH)])
        if zrem:
            rs = pl.ds(base + zfull * CH, zrem)
            pltpu.sync_copy(acc_sh.at[rs], rows_v.at[0, pl.ds(0, zrem)])
            pltpu.sync_copy(rows_v.at[0, pl.ds(0, zrem)], acc_out.at[ci, rs])
            pltpu.sync_copy(den_sh.at[rs], w_v.at[pl.ds(0, zrem)])
            pltpu.sync_copy(w_v.at[pl.ds(0, zrem)],
                            den_out.at[pl.ds(dbase + zfull * CH, zrem)])

    return sc_edge


def kernel(x, edge_index, W1, a_src1, a_dst1, b1, W2, a_src2, a_dst2, b2):
    n, d = x.shape
    dh = W1.shape[1]
    e = edge_index.shape[1]
    e2 = e + n

    rpt = -(-(n + 1) // NS)
    rpt = ((rpt + 7) // 8) * 8
    n_pad = rpt * NS
    per = NW * CH
    chunks = -(-e2 // per)
    chunks = -(-chunks // NBUF) * NBUF
    tot = NW * chunks * CH

    sl = jnp.arange(n, dtype=edge_index.dtype)
    src = jnp.concatenate([edge_index[0], sl,
                           jnp.zeros((tot - e2,), edge_index.dtype)])
    dst = jnp.concatenate([edge_index[1], sl,
                           jnp.full((tot - e2,), n, edge_index.dtype)])
    src_r = src.reshape(NW, chunks, CH)
    dst_r = dst.reshape(NW, chunks, CH)

    sc_edge = _make_sc_edge_kernel(n_pad, dh, chunks)

    def pad_tab(a):
        return jnp.pad(a[:, 0], (0, n_pad - n))

    def m_vec(m):
        s = m[0, 0] + m[0, 1]
        return jnp.full((LANES,), jnp.where(s >= 0, s, 0.2 * s))

    h1, as1, ad1, m1 = _tc_prep(x, W1, a_src1, a_dst1, blk=1000)
    acc1, den1 = sc_edge(src_r, dst_r, pad_tab(as1), pad_tab(ad1),
                         m_vec(m1), h1)

    h2, as2, ad2, m2 = _tc_mid(acc1, den1.reshape(NC, n_pad, 1),
                               b1.reshape(1, -1), W2, a_src2, a_dst2,
                               n, blk=1000)
    acc2, den2 = sc_edge(src_r, dst_r, pad_tab(as2), pad_tab(ad2),
                         m_vec(m2), h2)

    return _tc_fin(acc2, den2.reshape(NC, n_pad, 1),
                   b2.reshape(1, -1), n, blk=1000)


# speedup vs baseline: 1.1528x; 1.1528x over previous
import functools

import jax
import jax.numpy as jnp
from jax import lax
from jax.experimental import pallas as pl
from jax.experimental.pallas import tpu as pltpu
from jax.experimental.pallas import tpu_sc as plsc

NC = 2
NS = 16
NW = NC * NS
LANES = 16
CH = 96
NBUF = 3


def _prep_body(x_ref, w_ref, asr_ref, adr_ref, h_ref, as_ref, ad_ref, m_ref):
    h = jnp.dot(x_ref[...], w_ref[...], preferred_element_type=jnp.float32)
    h_ref[...] = h
    a_s = jnp.sum(h * asr_ref[...], axis=1, keepdims=True)
    a_d = jnp.sum(h * adr_ref[...], axis=1, keepdims=True)
    as_ref[...] = a_s
    ad_ref[...] = a_d

    @pl.when(pl.program_id(0) == 0)
    def _():
        m_ref[0, 0] = -jnp.inf
        m_ref[0, 1] = -jnp.inf

    m_ref[0, 0] = jnp.maximum(m_ref[0, 0], jnp.max(a_s))
    m_ref[0, 1] = jnp.maximum(m_ref[0, 1], jnp.max(a_d))


def _mid_body(acc_ref, den_ref, b_ref, w_ref, asr_ref, adr_ref,
              h_ref, as_ref, ad_ref, m_ref):
    a = acc_ref[0] + acc_ref[1]
    d = den_ref[0] + den_ref[1] + 1e-16
    out1 = a / d + b_ref[...]
    z = jnp.where(out1 > 0, out1, jnp.exp(jnp.minimum(out1, 0.0)) - 1.0)
    h = jnp.dot(z, w_ref[...], preferred_element_type=jnp.float32)
    h_ref[...] = h
    a_s = jnp.sum(h * asr_ref[...], axis=1, keepdims=True)
    a_d = jnp.sum(h * adr_ref[...], axis=1, keepdims=True)
    as_ref[...] = a_s
    ad_ref[...] = a_d

    @pl.when(pl.program_id(0) == 0)
    def _():
        m_ref[0, 0] = -jnp.inf
        m_ref[0, 1] = -jnp.inf

    m_ref[0, 0] = jnp.maximum(m_ref[0, 0], jnp.max(a_s))
    m_ref[0, 1] = jnp.maximum(m_ref[0, 1], jnp.max(a_d))


def _fin_body(acc_ref, den_ref, b_ref, out_ref):
    a = acc_ref[0] + acc_ref[1]
    d = den_ref[0] + den_ref[1] + 1e-16
    out_ref[...] = a / d + b_ref[...]


def _tc_prep(x, W, a_src, a_dst, blk):
    n, dh = x.shape[0], W.shape[1]
    grid = n // blk
    return pl.pallas_call(
        _prep_body,
        grid=(grid,),
        in_specs=[
            pl.BlockSpec((blk, x.shape[1]), lambda i: (i, 0)),
            pl.BlockSpec(W.shape, lambda i: (0, 0)),
            pl.BlockSpec(a_src.shape, lambda i: (0, 0)),
            pl.BlockSpec(a_dst.shape, lambda i: (0, 0)),
        ],
        out_specs=[
            pl.BlockSpec((blk, dh), lambda i: (i, 0)),
            pl.BlockSpec((blk, 1), lambda i: (i, 0)),
            pl.BlockSpec((blk, 1), lambda i: (i, 0)),
            pl.BlockSpec((1, 2), lambda i: (0, 0), memory_space=pltpu.SMEM),
        ],
        out_shape=[
            jax.ShapeDtypeStruct((n, dh), jnp.float32),
            jax.ShapeDtypeStruct((n, 1), jnp.float32),
            jax.ShapeDtypeStruct((n, 1), jnp.float32),
            jax.ShapeDtypeStruct((1, 2), jnp.float32),
        ],
    )(x, W, a_src, a_dst)


def _tc_mid(acc, den, b, W, a_src, a_dst, n, blk):
    dh = W.shape[1]
    grid = n // blk
    return pl.pallas_call(
        _mid_body,
        grid=(grid,),
        in_specs=[
            pl.BlockSpec((2, blk, acc.shape[2]), lambda i: (0, i, 0)),
            pl.BlockSpec((2, blk, 1), lambda i: (0, i, 0)),
            pl.BlockSpec((1, b.shape[1]), lambda i: (0, 0)),
            pl.BlockSpec(W.shape, lambda i: (0, 0)),
            pl.BlockSpec(a_src.shape, lambda i: (0, 0)),
            pl.BlockSpec(a_dst.shape, lambda i: (0, 0)),
        ],
        out_specs=[
            pl.BlockSpec((blk, dh), lambda i: (i, 0)),
            pl.BlockSpec((blk, 1), lambda i: (i, 0)),
            pl.BlockSpec((blk, 1), lambda i: (i, 0)),
            pl.BlockSpec((1, 2), lambda i: (0, 0), memory_space=pltpu.SMEM),
        ],
        out_shape=[
            jax.ShapeDtypeStruct((n, dh), jnp.float32),
            jax.ShapeDtypeStruct((n, 1), jnp.float32),
            jax.ShapeDtypeStruct((n, 1), jnp.float32),
            jax.ShapeDtypeStruct((1, 2), jnp.float32),
        ],
    )(acc, den, b, W, a_src, a_dst)


def _tc_fin(acc, den, b, n, blk):
    dh = acc.shape[2]
    grid = n // blk
    return pl.pallas_call(
        _fin_body,
        grid=(grid,),
        in_specs=[
            pl.BlockSpec((2, blk, dh), lambda i: (0, i, 0)),
            pl.BlockSpec((2, blk, 1), lambda i: (0, i, 0)),
            pl.BlockSpec((1, b.shape[1]), lambda i: (0, 0)),
        ],
        out_specs=pl.BlockSpec((blk, dh), lambda i: (i, 0)),
        out_shape=jax.ShapeDtypeStruct((n, dh), jnp.float32),
    )(acc, den, b)


@functools.lru_cache(maxsize=None)
def _make_sc_edge_kernel(n_pad, dh, chunks):
    mesh = plsc.VectorSubcoreMesh(
        core_axis_name="c", subcore_axis_name="s",
        num_cores=NC, num_subcores=NS)
    rows_per_tile = n_pad // NS
    zfull = rows_per_tile // CH
    zrem = rows_per_tile - zfull * CH

    @functools.partial(
        pl.kernel,
        out_type=(
            jax.ShapeDtypeStruct((NC, n_pad, dh), jnp.float32),
            jax.ShapeDtypeStruct((NC * n_pad,), jnp.float32),
        ),
        mesh=mesh,
        compiler_params=pltpu.CompilerParams(needs_layout_passes=False),
        scratch_types=[
            pltpu.VMEM((NBUF, CH), jnp.int32),
            pltpu.VMEM((NBUF, CH), jnp.int32),
            pltpu.VMEM((NBUF, CH), jnp.float32),
            pltpu.VMEM((NBUF, CH), jnp.float32),
            pltpu.VMEM((LANES,), jnp.float32),
            pltpu.VMEM((NBUF, CH, dh), jnp.float32),
            pltpu.VMEM((CH,), jnp.float32),
            pltpu.VMEM_SHARED((n_pad, dh), jnp.float32),
            pltpu.VMEM_SHARED((n_pad,), jnp.float32),
        ] + [pltpu.SemaphoreType.DMA] * (2 * NBUF),
    )
    def sc_edge(src_hbm, dst_hbm, as_hbm, ad_hbm, m_hbm, h_hbm,
                acc_out, den_out,
                src_i, dst_i, asg_v, adg_v, m_v, rows_v, w_v,
                acc_sh, den_sh, *sems):
        ci = lax.axis_index("c")
        si = lax.axis_index("s")
        wid = ci * NS + si
        semg = sems[:NBUF]
        semi = sems[NBUF:]

        pltpu.sync_copy(m_hbm, m_v)

        zero16 = jnp.zeros((LANES,), jnp.float32)

        def zrow(j, _):
            for r in range(dh // LANES):
                rows_v[0, j, pl.ds(r * LANES, LANES)] = zero16
            return _

        lax.fori_loop(0, CH, zrow, None)
        for r in range(CH // LANES):
            w_v[pl.ds(r * LANES, LANES)] = zero16
        base = si * rows_per_tile
        for t in range(zfull):
            pltpu.sync_copy(rows_v.at[0], acc_sh.at[pl.ds(base + t * CH, CH)])
            pltpu.sync_copy(w_v, den_sh.at[pl.ds(base + t * CH, CH)])
        if zrem:
            rs = pl.ds(base + zfull * CH, zrem)
            pltpu.sync_copy(rows_v.at[0, pl.ds(0, zrem)], acc_sh.at[rs])
            pltpu.sync_copy(w_v.at[pl.ds(0, zrem)], den_sh.at[rs])
        plsc.subcore_barrier()

        m_vec = m_v[...]

        for k in range(NBUF - 1):
            pltpu.sync_copy(src_hbm.at[wid, k], src_i.at[k])
            pltpu.sync_copy(dst_hbm.at[wid, k], dst_i.at[k])
            pltpu.async_copy(h_hbm.at[src_i.at[k]], rows_v.at[k], semg[k])
            pltpu.async_copy(as_hbm.at[src_i.at[k]], asg_v.at[k], semg[k])
            pltpu.async_copy(ad_hbm.at[dst_i.at[k]], adg_v.at[k], semg[k])
        pltpu.async_copy(src_hbm.at[wid, NBUF - 1], src_i.at[NBUF - 1],
                         semi[NBUF - 1])
        pltpu.async_copy(dst_hbm.at[wid, NBUF - 1], dst_i.at[NBUF - 1],
                         semi[NBUF - 1])

        def compute_w(b):
            for g in range(CH // LANES):
                sl = pl.ds(g * LANES, LANES)
                e = asg_v[b, sl] + adg_v[b, sl]
                e = jnp.where(e >= 0, e, e * 0.2) - m_vec
                w_v[sl] = jnp.exp(e)

        def group(g, _):
            for b in range(NBUF):
                j = g * NBUF + b
                t = (b + NBUF - 1) % NBUF

                @pl.when(j + NBUF - 1 < chunks)
                def _():
                    pltpu.make_async_copy(
                        src_hbm.at[wid, j + NBUF - 1], src_i.at[t],
                        semi[t]).wait()
                    pltpu.make_async_copy(
                        dst_hbm.at[wid, j + NBUF - 1], dst_i.at[t],
                        semi[t]).wait()
                    pltpu.async_copy(
                        h_hbm.at[src_i.at[t]], rows_v.at[t], semg[t])
                    pltpu.async_copy(
                        as_hbm.at[src_i.at[t]], asg_v.at[t], semg[t])
                    pltpu.async_copy(
                        ad_hbm.at[dst_i.at[t]], adg_v.at[t], semg[t])

                pltpu.make_async_copy(
                    h_hbm.at[src_i.at[b]], rows_v.at[b], semg[b]).wait()
                pltpu.make_async_copy(
                    as_hbm.at[src_i.at[b]], asg_v.at[b], semg[b]).wait()
                pltpu.make_async_copy(
                    ad_hbm.at[dst_i.at[b]], adg_v.at[b], semg[b]).wait()
                compute_w(b)

                def scale(g2, _c):
                    w16 = w_v[pl.ds(g2 * LANES, LANES)]
                    for jj in range(LANES):
                        w = jnp.full((LANES,), w16[jj])
                        for r in range(dh // LANES):
                            sl = pl.ds(r * LANES, LANES)
                            rows_v[b, g2 * LANES + jj, sl] = (
                                rows_v[b, g2 * LANES + jj, sl] * w)
                    return _c

                lax.fori_loop(0, CH // LANES, scale, None)
                pltpu.sync_copy(rows_v.at[b], acc_sh.at[dst_i.at[b]], add=True)
                pltpu.sync_copy(w_v, den_sh.at[dst_i.at[b]], add=True)

                @pl.when(j + NBUF < chunks)
                def _():
                    pltpu.async_copy(src_hbm.at[wid, j + NBUF], src_i.at[b],
                                     semi[b])
                    pltpu.async_copy(dst_hbm.at[wid, j + NBUF], dst_i.at[b],
                                     semi[b])
            return _

        lax.fori_loop(0, chunks // NBUF, group, None)
        plsc.subcore_barrier()

        dbase = pl.multiple_of(ci * n_pad + base, 8)
        for t in range(zfull):
            rs = pl.ds(base + t * CH, CH)
            pltpu.sync_copy(acc_sh.at[rs], rows_v.at[0])
            pltpu.sync_copy(rows_v.at[0], acc_out.at[ci, rs])
            pltpu.sync_copy(den_sh.at[rs], w_v)
            pltpu.sync_copy(w_v, den_out.at[pl.ds(dbase + t * CH, CH)])
        if zrem:
            rs = pl.ds(base + zfull * CH, zrem)
            pltpu.sync_copy(acc_sh.at[rs], rows_v.at[0, pl.ds(0, zrem)])
            pltpu.sync_copy(rows_v.at[0, pl.ds(0, zrem)], acc_out.at[ci, rs])
            pltpu.sync_copy(den_sh.at[rs], w_v.at[pl.ds(0, zrem)])
            pltpu.sync_copy(w_v.at[pl.ds(0, zrem)],
                            den_out.at[pl.ds(dbase + zfull * CH, zrem)])

    return sc_edge


def kernel(x, edge_index, W1, a_src1, a_dst1, b1, W2, a_src2, a_dst2, b2):
    n, d = x.shape
    dh = W1.shape[1]
    e = edge_index.shape[1]
    e2 = e + n

    rpt = -(-(n + 1) // NS)
    rpt = ((rpt + 7) // 8) * 8
    n_pad = rpt * NS
    per = NW * CH
    chunks = -(-e2 // per)
    chunks = -(-chunks // NBUF) * NBUF
    tot = NW * chunks * CH

    sl = jnp.arange(n, dtype=edge_index.dtype)
    src = jnp.concatenate([edge_index[0], sl,
                           jnp.zeros((tot - e2,), edge_index.dtype)])
    dst = jnp.concatenate([edge_index[1], sl,
                           jnp.full((tot - e2,), n, edge_index.dtype)])
    src_r = src.reshape(NW, chunks, CH)
    dst_r = dst.reshape(NW, chunks, CH)

    sc_edge = _make_sc_edge_kernel(n_pad, dh, chunks)

    def pad_tab(a):
        return jnp.pad(a[:, 0], (0, n_pad - n))

    def m_vec(m):
        s = m[0, 0] + m[0, 1]
        return jnp.full((LANES,), jnp.where(s >= 0, s, 0.2 * s))

    h1, as1, ad1, m1 = _tc_prep(x, W1, a_src1, a_dst1, blk=1000)
    acc1, den1 = sc_edge(src_r, dst_r, pad_tab(as1), pad_tab(ad1),
                         m_vec(m1), h1)

    h2, as2, ad2, m2 = _tc_mid(acc1, den1.reshape(NC, n_pad, 1),
                               b1.reshape(1, -1), W2, a_src2, a_dst2,
                               n, blk=1000)
    acc2, den2 = sc_edge(src_r, dst_r, pad_tab(as2), pad_tab(ad2),
                         m_vec(m2), h2)

    return _tc_fin(acc2, den2.reshape(NC, n_pad, 1),
                   b2.reshape(1, -1), n, blk=1000)


# speedup vs baseline: 1.1602x; 1.0064x over previous
import functools

import jax
import jax.numpy as jnp
from jax import lax
from jax.experimental import pallas as pl
from jax.experimental.pallas import tpu as pltpu
from jax.experimental.pallas import tpu_sc as plsc

NC = 2
NS = 16
NW = NC * NS
LANES = 16
CH = 96
NBUF = 3


def _prep_body(x_ref, w_ref, asr_ref, adr_ref, h_ref, as_ref, ad_ref, m_ref):
    h = jnp.dot(x_ref[...], w_ref[...], preferred_element_type=jnp.float32)
    h_ref[...] = h
    a_s = jnp.sum(h * asr_ref[...], axis=1, keepdims=True)
    a_d = jnp.sum(h * adr_ref[...], axis=1, keepdims=True)
    as_ref[...] = a_s
    ad_ref[...] = a_d

    @pl.when(pl.program_id(0) == 0)
    def _():
        m_ref[0, 0] = -jnp.inf
        m_ref[0, 1] = -jnp.inf

    m_ref[0, 0] = jnp.maximum(m_ref[0, 0], jnp.max(a_s))
    m_ref[0, 1] = jnp.maximum(m_ref[0, 1], jnp.max(a_d))


def _mid_body(acc_ref, den_ref, b_ref, w_ref, asr_ref, adr_ref,
              h_ref, as_ref, ad_ref, m_ref):
    a = acc_ref[0] + acc_ref[1]
    d = den_ref[0] + den_ref[1] + 1e-16
    out1 = a / d + b_ref[...]
    z = jnp.where(out1 > 0, out1, jnp.exp(jnp.minimum(out1, 0.0)) - 1.0)
    h = jnp.dot(z, w_ref[...], preferred_element_type=jnp.float32)
    h_ref[...] = h
    a_s = jnp.sum(h * asr_ref[...], axis=1, keepdims=True)
    a_d = jnp.sum(h * adr_ref[...], axis=1, keepdims=True)
    as_ref[...] = a_s
    ad_ref[...] = a_d

    @pl.when(pl.program_id(0) == 0)
    def _():
        m_ref[0, 0] = -jnp.inf
        m_ref[0, 1] = -jnp.inf

    m_ref[0, 0] = jnp.maximum(m_ref[0, 0], jnp.max(a_s))
    m_ref[0, 1] = jnp.maximum(m_ref[0, 1], jnp.max(a_d))


def _fin_body(acc_ref, den_ref, b_ref, out_ref):
    a = acc_ref[0] + acc_ref[1]
    d = den_ref[0] + den_ref[1] + 1e-16
    out_ref[...] = a / d + b_ref[...]


def _tc_prep(x, W, a_src, a_dst, blk):
    n, dh = x.shape[0], W.shape[1]
    grid = n // blk
    return pl.pallas_call(
        _prep_body,
        grid=(grid,),
        in_specs=[
            pl.BlockSpec((blk, x.shape[1]), lambda i: (i, 0)),
            pl.BlockSpec(W.shape, lambda i: (0, 0)),
            pl.BlockSpec(a_src.shape, lambda i: (0, 0)),
            pl.BlockSpec(a_dst.shape, lambda i: (0, 0)),
        ],
        out_specs=[
            pl.BlockSpec((blk, dh), lambda i: (i, 0)),
            pl.BlockSpec((blk, 1), lambda i: (i, 0)),
            pl.BlockSpec((blk, 1), lambda i: (i, 0)),
            pl.BlockSpec((1, 2), lambda i: (0, 0), memory_space=pltpu.SMEM),
        ],
        out_shape=[
            jax.ShapeDtypeStruct((n, dh), jnp.float32),
            jax.ShapeDtypeStruct((n, 1), jnp.float32),
            jax.ShapeDtypeStruct((n, 1), jnp.float32),
            jax.ShapeDtypeStruct((1, 2), jnp.float32),
        ],
    )(x, W, a_src, a_dst)


def _tc_mid(acc, den, b, W, a_src, a_dst, n, blk):
    dh = W.shape[1]
    grid = n // blk
    return pl.pallas_call(
        _mid_body,
        grid=(grid,),
        in_specs=[
            pl.BlockSpec((2, blk, acc.shape[2]), lambda i: (0, i, 0)),
            pl.BlockSpec((2, blk, 1), lambda i: (0, i, 0)),
            pl.BlockSpec((1, b.shape[1]), lambda i: (0, 0)),
            pl.BlockSpec(W.shape, lambda i: (0, 0)),
            pl.BlockSpec(a_src.shape, lambda i: (0, 0)),
            pl.BlockSpec(a_dst.shape, lambda i: (0, 0)),
        ],
        out_specs=[
            pl.BlockSpec((blk, dh), lambda i: (i, 0)),
            pl.BlockSpec((blk, 1), lambda i: (i, 0)),
            pl.BlockSpec((blk, 1), lambda i: (i, 0)),
            pl.BlockSpec((1, 2), lambda i: (0, 0), memory_space=pltpu.SMEM),
        ],
        out_shape=[
            jax.ShapeDtypeStruct((n, dh), jnp.float32),
            jax.ShapeDtypeStruct((n, 1), jnp.float32),
            jax.ShapeDtypeStruct((n, 1), jnp.float32),
            jax.ShapeDtypeStruct((1, 2), jnp.float32),
        ],
    )(acc, den, b, W, a_src, a_dst)


def _tc_fin(acc, den, b, n, blk):
    dh = acc.shape[2]
    grid = n // blk
    return pl.pallas_call(
        _fin_body,
        grid=(grid,),
        in_specs=[
            pl.BlockSpec((2, blk, dh), lambda i: (0, i, 0)),
            pl.BlockSpec((2, blk, 1), lambda i: (0, i, 0)),
            pl.BlockSpec((1, b.shape[1]), lambda i: (0, 0)),
        ],
        out_specs=pl.BlockSpec((blk, dh), lambda i: (i, 0)),
        out_shape=jax.ShapeDtypeStruct((n, dh), jnp.float32),
    )(acc, den, b)


@functools.lru_cache(maxsize=None)
def _make_sc_edge_kernel(n_pad, dh, chunks):
    mesh = plsc.VectorSubcoreMesh(
        core_axis_name="c", subcore_axis_name="s",
        num_cores=NC, num_subcores=NS)
    rows_per_tile = n_pad // NS
    zfull = rows_per_tile // CH
    zrem = rows_per_tile - zfull * CH

    @functools.partial(
        pl.kernel,
        out_type=(
            jax.ShapeDtypeStruct((NC, n_pad, dh), jnp.float32),
            jax.ShapeDtypeStruct((NC * n_pad,), jnp.float32),
        ),
        mesh=mesh,
        compiler_params=pltpu.CompilerParams(needs_layout_passes=False),
        scratch_types=[
            pltpu.VMEM((NBUF, CH), jnp.int32),
            pltpu.VMEM((NBUF, CH), jnp.int32),
            pltpu.VMEM((NBUF, CH), jnp.float32),
            pltpu.VMEM((NBUF, CH), jnp.float32),
            pltpu.VMEM((LANES,), jnp.float32),
            pltpu.VMEM((NBUF, CH, dh), jnp.float32),
            pltpu.VMEM((CH,), jnp.float32),
            pltpu.VMEM_SHARED((n_pad, dh), jnp.float32),
            pltpu.VMEM_SHARED((n_pad,), jnp.float32),
        ] + [pltpu.SemaphoreType.DMA] * (2 * NBUF),
    )
    def sc_edge(src_hbm, dst_hbm, as_hbm, ad_hbm, m_hbm, h_hbm,
                acc_out, den_out,
                src_i, dst_i, asg_v, adg_v, m_v, rows_v, w_v,
                acc_sh, den_sh, *sems):
        ci = lax.axis_index("c")
        si = lax.axis_index("s")
        wid = ci * NS + si
        semg = sems[:NBUF]
        semi = sems[NBUF:]

        pltpu.sync_copy(m_hbm, m_v)

        zero16 = jnp.zeros((LANES,), jnp.float32)

        def zrow(j, _):
            for r in range(dh // LANES):
                rows_v[0, j, pl.ds(r * LANES, LANES)] = zero16
            return _

        lax.fori_loop(0, CH, zrow, None)
        for r in range(CH // LANES):
            w_v[pl.ds(r * LANES, LANES)] = zero16
        base = si * rows_per_tile
        for t in range(zfull):
            pltpu.sync_copy(rows_v.at[0], acc_sh.at[pl.ds(base + t * CH, CH)])
            pltpu.sync_copy(w_v, den_sh.at[pl.ds(base + t * CH, CH)])
        if zrem:
            rs = pl.ds(base + zfull * CH, zrem)
            pltpu.sync_copy(rows_v.at[0, pl.ds(0, zrem)], acc_sh.at[rs])
            pltpu.sync_copy(w_v.at[pl.ds(0, zrem)], den_sh.at[rs])
        plsc.subcore_barrier()

        m_vec = m_v[...]

        for k in range(NBUF - 1):
            pltpu.sync_copy(src_hbm.at[k, wid], src_i.at[k])
            pltpu.sync_copy(dst_hbm.at[k, wid], dst_i.at[k])
            pltpu.async_copy(h_hbm.at[src_i.at[k]], rows_v.at[k], semg[k])
            pltpu.async_copy(as_hbm.at[src_i.at[k]], asg_v.at[k], semg[k])
            pltpu.async_copy(ad_hbm.at[dst_i.at[k]], adg_v.at[k], semg[k])
        pltpu.async_copy(src_hbm.at[NBUF - 1, wid], src_i.at[NBUF - 1],
                         semi[NBUF - 1])
        pltpu.async_copy(dst_hbm.at[NBUF - 1, wid], dst_i.at[NBUF - 1],
                         semi[NBUF - 1])

        def compute_w(b):
            for g in range(CH // LANES):
                sl = pl.ds(g * LANES, LANES)
                e = asg_v[b, sl] + adg_v[b, sl]
                e = jnp.where(e >= 0, e, e * 0.2) - m_vec
                w_v[sl] = jnp.exp(e)

        def group(g, _):
            for b in range(NBUF):
                j = g * NBUF + b
                t = (b + NBUF - 1) % NBUF

                @pl.when(j + NBUF - 1 < chunks)
                def _():
                    pltpu.make_async_copy(
                        src_hbm.at[j + NBUF - 1, wid], src_i.at[t],
                        semi[t]).wait()
                    pltpu.make_async_copy(
                        dst_hbm.at[j + NBUF - 1, wid], dst_i.at[t],
                        semi[t]).wait()
                    pltpu.async_copy(
                        h_hbm.at[src_i.at[t]], rows_v.at[t], semg[t])
                    pltpu.async_copy(
                        as_hbm.at[src_i.at[t]], asg_v.at[t], semg[t])
                    pltpu.async_copy(
                        ad_hbm.at[dst_i.at[t]], adg_v.at[t], semg[t])

                pltpu.make_async_copy(
                    h_hbm.at[src_i.at[b]], rows_v.at[b], semg[b]).wait()
                pltpu.make_async_copy(
                    as_hbm.at[src_i.at[b]], asg_v.at[b], semg[b]).wait()
                pltpu.make_async_copy(
                    ad_hbm.at[dst_i.at[b]], adg_v.at[b], semg[b]).wait()
                compute_w(b)

                def scale(g2, _c):
                    w16 = w_v[pl.ds(g2 * LANES, LANES)]
                    for jj in range(LANES):
                        w = jnp.full((LANES,), w16[jj])
                        for r in range(dh // LANES):
                            sl = pl.ds(r * LANES, LANES)
                            rows_v[b, g2 * LANES + jj, sl] = (
                                rows_v[b, g2 * LANES + jj, sl] * w)
                    return _c

                lax.fori_loop(0, CH // LANES, scale, None)
                pltpu.sync_copy(rows_v.at[b], acc_sh.at[dst_i.at[b]], add=True)
                pltpu.sync_copy(w_v, den_sh.at[dst_i.at[b]], add=True)

                @pl.when(j + NBUF < chunks)
                def _():
                    pltpu.async_copy(src_hbm.at[j + NBUF, wid], src_i.at[b],
                                     semi[b])
                    pltpu.async_copy(dst_hbm.at[j + NBUF, wid], dst_i.at[b],
                                     semi[b])
            return _

        lax.fori_loop(0, chunks // NBUF, group, None)
        plsc.subcore_barrier()

        dbase = pl.multiple_of(ci * n_pad + base, 8)
        for t in range(zfull):
            rs = pl.ds(base + t * CH, CH)
            pltpu.sync_copy(acc_sh.at[rs], rows_v.at[0])
            pltpu.sync_copy(rows_v.at[0], acc_out.at[ci, rs])
            pltpu.sync_copy(den_sh.at[rs], w_v)
            pltpu.sync_copy(w_v, den_out.at[pl.ds(dbase + t * CH, CH)])
        if zrem:
            rs = pl.ds(base + zfull * CH, zrem)
            pltpu.sync_copy(acc_sh.at[rs], rows_v.at[0, pl.ds(0, zrem)])
            pltpu.sync_copy(rows_v.at[0, pl.ds(0, zrem)], acc_out.at[ci, rs])
            pltpu.sync_copy(den_sh.at[rs], w_v.at[pl.ds(0, zrem)])
            pltpu.sync_copy(w_v.at[pl.ds(0, zrem)],
                            den_out.at[pl.ds(dbase + zfull * CH, zrem)])

    return sc_edge


def kernel(x, edge_index, W1, a_src1, a_dst1, b1, W2, a_src2, a_dst2, b2):
    n, d = x.shape
    dh = W1.shape[1]
    e = edge_index.shape[1]
    e2 = e + n

    rpt = -(-(n + 1) // NS)
    rpt = ((rpt + 7) // 8) * 8
    n_pad = rpt * NS
    per = NW * CH
    chunks = -(-e2 // per)
    chunks = -(-chunks // NBUF) * NBUF
    tot = NW * chunks * CH

    sl = jnp.arange(n, dtype=edge_index.dtype)
    src = jnp.concatenate([edge_index[0], sl,
                           jnp.zeros((tot - e2,), edge_index.dtype)])
    dst = jnp.concatenate([edge_index[1], sl,
                           jnp.full((tot - e2,), n, edge_index.dtype)])
    src_r = src.reshape(chunks, NW, CH)
    dst_r = dst.reshape(chunks, NW, CH)

    sc_edge = _make_sc_edge_kernel(n_pad, dh, chunks)

    def pad_tab(a):
        return jnp.pad(a[:, 0], (0, n_pad - n))

    def m_vec(m):
        s = m[0, 0] + m[0, 1]
        return jnp.full((LANES,), jnp.where(s >= 0, s, 0.2 * s))

    h1, as1, ad1, m1 = _tc_prep(x, W1, a_src1, a_dst1, blk=1000)
    acc1, den1 = sc_edge(src_r, dst_r, pad_tab(as1), pad_tab(ad1),
                         m_vec(m1), h1)

    h2, as2, ad2, m2 = _tc_mid(acc1, den1.reshape(NC, n_pad, 1),
                               b1.reshape(1, -1), W2, a_src2, a_dst2,
                               n, blk=1000)
    acc2, den2 = sc_edge(src_r, dst_r, pad_tab(as2), pad_tab(ad2),
                         m_vec(m2), h2)

    return _tc_fin(acc2, den2.reshape(NC, n_pad, 1),
                   b2.reshape(1, -1), n, blk=1000)


# speedup vs baseline: 1.1716x; 1.0098x over previous
import functools

import jax
import jax.numpy as jnp
from jax import lax
from jax.experimental import pallas as pl
from jax.experimental.pallas import tpu as pltpu
from jax.experimental.pallas import tpu_sc as plsc

NC = 2
NS = 16
NW = NC * NS
LANES = 16
CH = 96
NBUF = 3


def _prep_body(x_ref, w_ref, asr_ref, adr_ref, h_ref, as_ref, ad_ref, m_ref):
    h = jnp.dot(x_ref[...], w_ref[...], preferred_element_type=jnp.float32)
    h_ref[...] = h
    a_s = jnp.sum(h * asr_ref[...], axis=1, keepdims=True)
    a_d = jnp.sum(h * adr_ref[...], axis=1, keepdims=True)
    as_ref[...] = a_s
    ad_ref[...] = a_d

    @pl.when(pl.program_id(0) == 0)
    def _():
        m_ref[0, 0] = -jnp.inf
        m_ref[0, 1] = -jnp.inf

    m_ref[0, 0] = jnp.maximum(m_ref[0, 0], jnp.max(a_s))
    m_ref[0, 1] = jnp.maximum(m_ref[0, 1], jnp.max(a_d))


def _mid_body(acc_ref, den_ref, b_ref, w_ref, asr_ref, adr_ref,
              h_ref, as_ref, ad_ref, m_ref):
    a = acc_ref[0] + acc_ref[1]
    d = den_ref[0] + den_ref[1] + 1e-16
    out1 = a / d + b_ref[...]
    z = jnp.where(out1 > 0, out1, jnp.exp(jnp.minimum(out1, 0.0)) - 1.0)
    h = jnp.dot(z, w_ref[...], preferred_element_type=jnp.float32)
    h_ref[...] = h
    a_s = jnp.sum(h * asr_ref[...], axis=1, keepdims=True)
    a_d = jnp.sum(h * adr_ref[...], axis=1, keepdims=True)
    as_ref[...] = a_s
    ad_ref[...] = a_d

    @pl.when(pl.program_id(0) == 0)
    def _():
        m_ref[0, 0] = -jnp.inf
        m_ref[0, 1] = -jnp.inf

    m_ref[0, 0] = jnp.maximum(m_ref[0, 0], jnp.max(a_s))
    m_ref[0, 1] = jnp.maximum(m_ref[0, 1], jnp.max(a_d))


def _fin_body(acc_ref, den_ref, b_ref, out_ref):
    a = acc_ref[0] + acc_ref[1]
    d = den_ref[0] + den_ref[1] + 1e-16
    out_ref[...] = a / d + b_ref[...]


def _tc_prep(x, W, a_src, a_dst, blk):
    n, dh = x.shape[0], W.shape[1]
    grid = n // blk
    return pl.pallas_call(
        _prep_body,
        grid=(grid,),
        in_specs=[
            pl.BlockSpec((blk, x.shape[1]), lambda i: (i, 0)),
            pl.BlockSpec(W.shape, lambda i: (0, 0)),
            pl.BlockSpec(a_src.shape, lambda i: (0, 0)),
            pl.BlockSpec(a_dst.shape, lambda i: (0, 0)),
        ],
        out_specs=[
            pl.BlockSpec((blk, dh), lambda i: (i, 0)),
            pl.BlockSpec((blk, 1), lambda i: (i, 0)),
            pl.BlockSpec((blk, 1), lambda i: (i, 0)),
            pl.BlockSpec((1, 2), lambda i: (0, 0), memory_space=pltpu.SMEM),
        ],
        out_shape=[
            jax.ShapeDtypeStruct((n, dh), jnp.float32),
            jax.ShapeDtypeStruct((n, 1), jnp.float32),
            jax.ShapeDtypeStruct((n, 1), jnp.float32),
            jax.ShapeDtypeStruct((1, 2), jnp.float32),
        ],
    )(x, W, a_src, a_dst)


def _tc_mid(acc, den, b, W, a_src, a_dst, n, blk):
    dh = W.shape[1]
    grid = n // blk
    return pl.pallas_call(
        _mid_body,
        grid=(grid,),
        in_specs=[
            pl.BlockSpec((2, blk, acc.shape[2]), lambda i: (0, i, 0)),
            pl.BlockSpec((2, blk, 1), lambda i: (0, i, 0)),
            pl.BlockSpec((1, b.shape[1]), lambda i: (0, 0)),
            pl.BlockSpec(W.shape, lambda i: (0, 0)),
            pl.BlockSpec(a_src.shape, lambda i: (0, 0)),
            pl.BlockSpec(a_dst.shape, lambda i: (0, 0)),
        ],
        out_specs=[
            pl.BlockSpec((blk, dh), lambda i: (i, 0)),
            pl.BlockSpec((blk, 1), lambda i: (i, 0)),
            pl.BlockSpec((blk, 1), lambda i: (i, 0)),
            pl.BlockSpec((1, 2), lambda i: (0, 0), memory_space=pltpu.SMEM),
        ],
        out_shape=[
            jax.ShapeDtypeStruct((n, dh), jnp.float32),
            jax.ShapeDtypeStruct((n, 1), jnp.float32),
            jax.ShapeDtypeStruct((n, 1), jnp.float32),
            jax.ShapeDtypeStruct((1, 2), jnp.float32),
        ],
    )(acc, den, b, W, a_src, a_dst)


def _tc_fin(acc, den, b, n, blk):
    dh = acc.shape[2]
    grid = n // blk
    return pl.pallas_call(
        _fin_body,
        grid=(grid,),
        in_specs=[
            pl.BlockSpec((2, blk, dh), lambda i: (0, i, 0)),
            pl.BlockSpec((2, blk, 1), lambda i: (0, i, 0)),
            pl.BlockSpec((1, b.shape[1]), lambda i: (0, 0)),
        ],
        out_specs=pl.BlockSpec((blk, dh), lambda i: (i, 0)),
        out_shape=jax.ShapeDtypeStruct((n, dh), jnp.float32),
    )(acc, den, b)


@functools.lru_cache(maxsize=None)
def _make_sc_edge_kernel(n_pad, dh, chunks):
    mesh = plsc.VectorSubcoreMesh(
        core_axis_name="c", subcore_axis_name="s",
        num_cores=NC, num_subcores=NS)
    rows_per_tile = n_pad // NS
    zfull = rows_per_tile // CH
    zrem = rows_per_tile - zfull * CH

    @functools.partial(
        pl.kernel,
        out_type=(
            jax.ShapeDtypeStruct((NC, n_pad, dh), jnp.float32),
            jax.ShapeDtypeStruct((NC * n_pad,), jnp.float32),
        ),
        mesh=mesh,
        compiler_params=pltpu.CompilerParams(needs_layout_passes=False),
        scratch_types=[
            pltpu.VMEM((NBUF, CH), jnp.int32),
            pltpu.VMEM((NBUF, CH), jnp.int32),
            pltpu.VMEM((NBUF, CH), jnp.float32),
            pltpu.VMEM((NBUF, CH), jnp.float32),
            pltpu.VMEM((LANES,), jnp.float32),
            pltpu.VMEM((NBUF, CH, dh), jnp.float32),
            pltpu.VMEM((CH,), jnp.float32),
            pltpu.VMEM_SHARED((n_pad, dh), jnp.float32),
            pltpu.VMEM_SHARED((n_pad,), jnp.float32),
        ] + [pltpu.SemaphoreType.DMA] * (2 * NBUF),
    )
    def sc_edge(src_hbm, dst_hbm, as_hbm, ad_hbm, m_hbm, h_hbm,
                acc_out, den_out,
                src_i, dst_i, asg_v, adg_v, m_v, rows_v, w_v,
                acc_sh, den_sh, *sems):
        ci = lax.axis_index("c")
        si = lax.axis_index("s")
        wid = ci * NS + si
        semg = sems[:NBUF]
        semi = sems[NBUF:]

        pltpu.sync_copy(m_hbm, m_v)

        zero16 = jnp.zeros((LANES,), jnp.float32)

        def zrow(j, _):
            for r in range(dh // LANES):
                rows_v[0, j, pl.ds(r * LANES, LANES)] = zero16
            return _

        lax.fori_loop(0, CH, zrow, None)
        for r in range(CH // LANES):
            w_v[pl.ds(r * LANES, LANES)] = zero16
        base = si * rows_per_tile
        for t in range(zfull):
            pltpu.sync_copy(rows_v.at[0], acc_sh.at[pl.ds(base + t * CH, CH)])
            pltpu.sync_copy(w_v, den_sh.at[pl.ds(base + t * CH, CH)])
        if zrem:
            rs = pl.ds(base + zfull * CH, zrem)
            pltpu.sync_copy(rows_v.at[0, pl.ds(0, zrem)], acc_sh.at[rs])
            pltpu.sync_copy(w_v.at[pl.ds(0, zrem)], den_sh.at[rs])
        plsc.subcore_barrier()

        m_vec = m_v[...]

        for k in range(NBUF - 1):
            pltpu.sync_copy(src_hbm.at[k, wid], src_i.at[k])
            pltpu.sync_copy(dst_hbm.at[k, wid], dst_i.at[k])
            pltpu.async_copy(h_hbm.at[src_i.at[k]], rows_v.at[k], semg[k])
            pltpu.async_copy(as_hbm.at[src_i.at[k]], asg_v.at[k], semg[k])
            pltpu.async_copy(ad_hbm.at[dst_i.at[k]], adg_v.at[k], semg[k])
        pltpu.async_copy(src_hbm.at[NBUF - 1, wid], src_i.at[NBUF - 1],
                         semi[NBUF - 1])
        pltpu.async_copy(dst_hbm.at[NBUF - 1, wid], dst_i.at[NBUF - 1],
                         semi[NBUF - 1])

        def compute_w(b):
            for g in range(CH // LANES):
                sl = pl.ds(g * LANES, LANES)
                e = asg_v[b, sl] + adg_v[b, sl]
                e = jnp.where(e >= 0, e, e * 0.2) - m_vec
                w_v[sl] = jnp.exp(e)

        def group(g, _):
            for b in range(NBUF):
                j = g * NBUF + b
                t = (b + NBUF - 1) % NBUF

                @pl.when(j + NBUF - 1 < chunks)
                def _():
                    pltpu.make_async_copy(
                        src_hbm.at[j + NBUF - 1, wid], src_i.at[t],
                        semi[t]).wait()
                    pltpu.make_async_copy(
                        dst_hbm.at[j + NBUF - 1, wid], dst_i.at[t],
                        semi[t]).wait()
                    pltpu.async_copy(
                        h_hbm.at[src_i.at[t]], rows_v.at[t], semg[t])
                    pltpu.async_copy(
                        as_hbm.at[src_i.at[t]], asg_v.at[t], semg[t])
                    pltpu.async_copy(
                        ad_hbm.at[dst_i.at[t]], adg_v.at[t], semg[t])

                pltpu.make_async_copy(
                    h_hbm.at[src_i.at[b]], rows_v.at[b], semg[b]).wait()
                pltpu.make_async_copy(
                    as_hbm.at[src_i.at[b]], asg_v.at[b], semg[b]).wait()
                pltpu.make_async_copy(
                    ad_hbm.at[dst_i.at[b]], adg_v.at[b], semg[b]).wait()
                compute_w(b)

                def scale(g2, _c):
                    w16 = w_v[pl.ds(g2 * LANES, LANES)]
                    for jj in range(LANES):
                        w = jnp.full((LANES,), w16[jj])
                        for r in range(dh // LANES):
                            sl = pl.ds(r * LANES, LANES)
                            rows_v[b, g2 * LANES + jj, sl] = (
                                rows_v[b, g2 * LANES + jj, sl] * w)
                    return _c

                lax.fori_loop(0, CH // LANES, scale, None)
                pltpu.sync_copy(rows_v.at[b], acc_sh.at[dst_i.at[b]], add=True)
                pltpu.sync_copy(w_v, den_sh.at[dst_i.at[b]], add=True)

                @pl.when(j + NBUF < chunks)
                def _():
                    pltpu.async_copy(src_hbm.at[j + NBUF, wid], src_i.at[b],
                                     semi[b])
                    pltpu.async_copy(dst_hbm.at[j + NBUF, wid], dst_i.at[b],
                                     semi[b])
            return _

        lax.fori_loop(0, chunks // NBUF, group, None)
        plsc.subcore_barrier()

        dbase = pl.multiple_of(ci * n_pad + base, 8)
        for t in range(zfull):
            rs = pl.ds(base + t * CH, CH)
            pltpu.sync_copy(acc_sh.at[rs], rows_v.at[0])
            pltpu.sync_copy(rows_v.at[0], acc_out.at[ci, rs])
            pltpu.sync_copy(den_sh.at[rs], w_v)
            pltpu.sync_copy(w_v, den_out.at[pl.ds(dbase + t * CH, CH)])
        if zrem:
            rs = pl.ds(base + zfull * CH, zrem)
            pltpu.sync_copy(acc_sh.at[rs], rows_v.at[0, pl.ds(0, zrem)])
            pltpu.sync_copy(rows_v.at[0, pl.ds(0, zrem)], acc_out.at[ci, rs])
            pltpu.sync_copy(den_sh.at[rs], w_v.at[pl.ds(0, zrem)])
            pltpu.sync_copy(w_v.at[pl.ds(0, zrem)],
                            den_out.at[pl.ds(dbase + zfull * CH, zrem)])

    return sc_edge


def kernel(x, edge_index, W1, a_src1, a_dst1, b1, W2, a_src2, a_dst2, b2):
    n, d = x.shape
    dh = W1.shape[1]
    e = edge_index.shape[1]
    e2 = e + n

    rpt = -(-(n + 1) // NS)
    rpt = ((rpt + 7) // 8) * 8
    n_pad = rpt * NS
    per = NW * CH
    chunks = -(-e2 // per)
    chunks = -(-chunks // NBUF) * NBUF
    tot = NW * chunks * CH

    sl = jnp.arange(n, dtype=edge_index.dtype)
    src = jnp.concatenate([edge_index[0], sl,
                           jnp.zeros((tot - e2,), edge_index.dtype)])
    dst = jnp.concatenate([edge_index[1], sl,
                           jnp.full((tot - e2,), n, edge_index.dtype)])
    src_r = src.reshape(chunks, NW, CH)
    dst_r = dst.reshape(chunks, NW, CH)

    sc_edge = _make_sc_edge_kernel(n_pad, dh, chunks)

    def pad_tab(a):
        return jnp.pad(a[:, 0], (0, n_pad - n))

    def m_vec(m):
        s = m[0, 0] + m[0, 1]
        return jnp.full((LANES,), jnp.where(s >= 0, s, 0.2 * s))

    h1, as1, ad1, m1 = _tc_prep(x, W1, a_src1, a_dst1, blk=2000)
    acc1, den1 = sc_edge(src_r, dst_r, pad_tab(as1), pad_tab(ad1),
                         m_vec(m1), h1)

    h2, as2, ad2, m2 = _tc_mid(acc1, den1.reshape(NC, n_pad, 1),
                               b1.reshape(1, -1), W2, a_src2, a_dst2,
                               n, blk=2000)
    acc2, den2 = sc_edge(src_r, dst_r, pad_tab(as2), pad_tab(ad2),
                         m_vec(m2), h2)

    return _tc_fin(acc2, den2.reshape(NC, n_pad, 1),
                   b2.reshape(1, -1), n, blk=2000)
